# baseline (speedup 1.0000x reference)
import functools

import jax
import jax.numpy as jnp
from jax import lax
from jax.experimental import pallas as pl
from jax.experimental.pallas import tpu as pltpu
from jax.experimental.pallas import tpu_sc as plsc

_N = 100000
_E = 1600000
_NS = 16
_CHUNK = 6400
_NCHUNK = _E // _CHUNK
_SCHED = 16
_GROUPS = _CHUNK // 16
_U = 8


def _make_kernel():
    mesh = plsc.VectorSubcoreMesh(core_axis_name="c", subcore_axis_name="s")

    @functools.partial(
        pl.kernel,
        out_type=jax.ShapeDtypeStruct((32, 16), jnp.float32),
        mesh=mesh,
        compiler_params=pltpu.CompilerParams(needs_layout_passes=False),
        scratch_types=[
            pltpu.VMEM((_N,), jnp.float32),
            pltpu.VMEM((2, 2, _CHUNK), jnp.int32),
            pltpu.VMEM((16,), jnp.float32),
            pltpu.SemaphoreType.DMA,
            pltpu.SemaphoreType.DMA((2,)),
        ],
    )
    def k(node_x, node_y, h_edges, v_edges, out,
          table_v, idx_v, acc_v, tsem, sems):
        cid = lax.axis_index("c")
        sid = lax.axis_index("s")
        c_lo = (_NCHUNK * sid) // _NS
        c_hi = (_NCHUNK * (sid + 1)) // _NS

        def phase(nodes_hbm, edges_hbm):
            def src(g):
                c = jnp.minimum(c_lo + g, c_hi - 1)
                off = pl.multiple_of(c * _CHUNK, 128)
                return edges_hbm.at[:, pl.ds(off, _CHUNK)]

            def start(b, g):
                pltpu.async_copy(src(g), idx_v.at[b], sems.at[b])

            def wait(b):
                pltpu.make_async_copy(src(0), idx_v.at[b], sems.at[b]).wait()

            def compute(b, g, accs):
                @plsc.parallel_loop(0, _GROUPS, unroll=_U,
                                    carry=(jnp.zeros((16,), jnp.float32),
                                           jnp.zeros((16,), jnp.float32)))
                def csum(j, cc):
                    c0, c1 = cc
                    s = pl.multiple_of(j * 16, 16)
                    a = plsc.load_gather(table_v, [idx_v[b, 0, pl.ds(s, 16)]])
                    bb = plsc.load_gather(table_v, [idx_v[b, 1, pl.ds(s, 16)]])
                    return (c1, c0 + jnp.abs(a - bb))

                live = c_lo + g < c_hi
                a0, a1 = accs
                s0, s1 = csum
                return (a0 + jnp.where(live, s0, 0.0),
                        a1 + jnp.where(live, s1, 0.0))

            tcp = pltpu.async_copy(nodes_hbm, table_v, tsem)
            start(0, 0)
            tcp.wait()

            def body(g, accs):
                b = lax.rem(g, 2)

                @pl.when(g + 1 < _SCHED)
                def _():
                    start(1 - b, g + 1)

                wait(b)
                return compute(b, g, accs)

            z = jnp.zeros((16,), jnp.float32)
            accs = lax.fori_loop(0, _SCHED, body, (z, z))
            acc_v[...] = accs[0] + accs[1]

        @pl.when(cid == 0)
        def _():
            phase(node_x, h_edges)

        @pl.when(cid == 1)
        def _():
            phase(node_y, v_edges)

        pltpu.sync_copy(acc_v, out.at[sid * 2 + cid])

    return k


_sc_kernel = _make_kernel()


def kernel(node_x, node_y, h_edges, v_edges):
    partials = _sc_kernel(node_x, node_y, h_edges, v_edges)
    return jnp.sum(partials)
